# baseline (speedup 1.0000x reference)
import functools

import jax
import jax.numpy as jnp
from jax import lax
from jax.experimental import pallas as pl
from jax.experimental.pallas import tpu as pltpu
from jax.experimental.pallas import tpu_sc as plsc

KCODES = 8192
DIM = 256
BETA = 0.25
KC = 1024

SC_CORES = 2
SC_SUBCORES = 16
NW = SC_CORES * SC_SUBCORES


def _assign_body(x_ref, ct_ref, qx_ref):
    hw = x_ref.shape[1]
    x_b = x_ref[0]
    xb16 = x_b.astype(jnp.bfloat16)
    xnorm = jnp.sum(x_b * x_b, axis=1, keepdims=True)
    rmin = jnp.full((hw, 1), jnp.inf, jnp.float32)
    ridx = jnp.zeros((hw, 1), jnp.int32)
    for j in range(KCODES // KC):
        ct = ct_ref[:, j * KC:(j + 1) * KC]
        cn = jnp.sum(ct * ct, axis=0, keepdims=True)
        prod = lax.dot_general(
            xb16, ct.astype(jnp.bfloat16),
            (((1,), (0,)), ((), ())),
            preferred_element_type=jnp.float32)
        d2 = (xnorm - 2.0 * prod) + cn
        s = jnp.sqrt(jnp.maximum(d2, 0.0))
        cmin = jnp.min(s, axis=1, keepdims=True)
        iot = lax.broadcasted_iota(jnp.int32, (hw, KC), 1) + (j * KC)
        cidx = jnp.min(jnp.where(s == cmin, iot, KCODES),
                       axis=1, keepdims=True)
        take = cmin < rmin
        rmin = jnp.where(take, cmin, rmin)
        ridx = jnp.where(take, cidx, ridx)
    qx_ref[0] = jnp.broadcast_to(ridx, (hw, 128))


def _finish_body(x_ref, xe_ref, out_ref, ls_ref):
    x_b = x_ref[0]
    xe_t = xe_ref[0].T
    d = x_b - xe_t
    out_ref[0] = x_b + d
    ls_ref[0] = jnp.broadcast_to(jnp.sum(d * d), (8, 128))


def _gather_rows(table, idx, n_rows):
    rows_per_w = n_rows // NW
    ch = 128
    n_ch = rows_per_w // ch
    mesh = plsc.VectorSubcoreMesh(core_axis_name="c", subcore_axis_name="s")

    @functools.partial(
        pl.kernel,
        out_type=jax.ShapeDtypeStruct((n_rows, DIM), jnp.float32),
        mesh=mesh,
        scratch_types=[
            pltpu.VMEM((ch,), jnp.int32),
            pltpu.VMEM((ch, DIM), jnp.float32),
            pltpu.SemaphoreType.DMA,
        ],
    )
    def k(table_hbm, idx_hbm, out_hbm, idx_v, rows_v, sem):
        wid = lax.axis_index("s") * SC_CORES + lax.axis_index("c")
        for t in range(n_ch):
            base = wid * rows_per_w + t * ch
            pltpu.sync_copy(idx_hbm.at[pl.ds(base, ch)], idx_v)
            pltpu.async_copy(table_hbm.at[idx_v], rows_v, sem).wait()
            pltpu.sync_copy(rows_v, out_hbm.at[pl.ds(base, ch)])

    return k(table, idx)


def kernel(x, lookup_table):
    b, d, h, w = x.shape
    hw = h * w
    n = b * hw
    x3 = x.reshape(b, d, hw)
    xrows = jnp.swapaxes(x3, 1, 2)
    ct = lookup_table.T

    qx3 = pl.pallas_call(
        _assign_body,
        grid=(b,),
        in_specs=[
            pl.BlockSpec((1, hw, DIM), lambda i: (i, 0, 0)),
            pl.BlockSpec((DIM, KCODES), lambda i: (0, 0)),
        ],
        out_specs=pl.BlockSpec((1, hw, 128), lambda i: (i, 0, 0)),
        out_shape=jax.ShapeDtypeStruct((b, hw, 128), jnp.int32),
    )(xrows, ct)
    qx3 = qx3[:, :, 0]

    xe_rows = _gather_rows(lookup_table, qx3.reshape(n), n)

    out3, ls = pl.pallas_call(
        _finish_body,
        grid=(b,),
        in_specs=[
            pl.BlockSpec((1, DIM, hw), lambda i: (i, 0, 0)),
            pl.BlockSpec((1, hw, DIM), lambda i: (i, 0, 0)),
        ],
        out_specs=[
            pl.BlockSpec((1, DIM, hw), lambda i: (i, 0, 0)),
            pl.BlockSpec((1, 8, 128), lambda i: (i, 0, 0)),
        ],
        out_shape=[
            jax.ShapeDtypeStruct((b, DIM, hw), jnp.float32),
            jax.ShapeDtypeStruct((b, 8, 128), jnp.float32),
        ],
    )(x3, xe_rows.reshape(b, hw, DIM))

    loss = (1.0 + BETA) * jnp.sum(ls[:, 0, 0]) / jnp.float32(x.size)
    return out3.reshape(b, d, h, w), qx3.reshape(b, h, w), loss
